# baseline (speedup 1.0000x reference)
import functools

import jax
import jax.numpy as jnp
from jax import lax
from jax.experimental import pallas as pl
from jax.experimental.pallas import tpu as pltpu
from jax.experimental.pallas import tpu_sc as plsc

VOCAB = 1000
EMB = 128
BATCH = 1024
SEQ = 128
ROWS = BATCH * SEQ
NCHUNK = 4
CROWS = ROWS // NCHUNK
CBATCH = BATCH // NCHUNK

_NC = 2
_NS = 16
_NW = _NC * _NS
_BPW = CROWS // _NW
_CH = 128
_NCH = _BPW // _CH


@functools.cache
def _make_sc_gather():
    mesh = plsc.VectorSubcoreMesh(core_axis_name="c", subcore_axis_name="s")

    @functools.partial(
        pl.kernel,
        mesh=mesh,
        out_type=jax.ShapeDtypeStruct((CROWS, EMB), jnp.float32),
        scratch_types=[
            pltpu.VMEM((_BPW,), jnp.int32),
            pltpu.VMEM((2, _CH, EMB), jnp.float32),
            pltpu.SemaphoreType.DMA,
            pltpu.SemaphoreType.DMA,
            pltpu.SemaphoreType.DMA,
        ],
    )
    def _sc_gather(idx_hbm, table_hbm, out_hbm, idx_v, rows_v, gsem, ws0, ws1):
        wid = lax.axis_index("s") * _NC + lax.axis_index("c")
        base = wid * _BPW
        pltpu.sync_copy(idx_hbm.at[pl.ds(base, _BPW)], idx_v)

        def body2(g, carry):
            for slot in (0, 1):
                c = 2 * g + slot
                off = pl.multiple_of(c * _CH, _CH)
                wsem = ws0 if slot == 0 else ws1

                @pl.when(g > 0)
                def _wait_prev():
                    pltpu.make_async_copy(
                        rows_v.at[slot], out_hbm.at[pl.ds(base, _CH)], wsem
                    ).wait()

                pltpu.async_copy(
                    table_hbm.at[idx_v.at[pl.ds(off, _CH)]],
                    rows_v.at[slot], gsem,
                ).wait()
                pltpu.async_copy(
                    rows_v.at[slot], out_hbm.at[pl.ds(base + off, _CH)], wsem
                )
            return carry

        lax.fori_loop(0, _NCH // 2, body2, 0)
        pltpu.make_async_copy(
            rows_v.at[0], out_hbm.at[pl.ds(base, _CH)], ws0
        ).wait()
        pltpu.make_async_copy(
            rows_v.at[1], out_hbm.at[pl.ds(base, _CH)], ws1
        ).wait()

    return _sc_gather


_NB = 32


def _tc_head_first(g_ref, wt_ref, b_ref, out_ref):
    wt = wt_ref[...]
    bb = b_ref[...]
    for nb in range(_NB):
        g = g_ref[nb].astype(jnp.bfloat16)
        acc = lax.dot_general(
            g, wt, (((0,), (0,)), ((), ())),
            preferred_element_type=jnp.float32,
        )
        out_ref[pl.ds(nb * EMB, EMB), :] = acc + bb


def _tc_head_next(g_ref, wt_ref, b_ref, prev_ref, out_ref):
    _tc_head_first(g_ref, wt_ref, b_ref, out_ref)


def kernel(idxs, token_emb, W, b):
    idx_flat = idxs.reshape(-1).astype(jnp.int32)
    wt = W.astype(jnp.bfloat16).T
    b2 = b.reshape(1, VOCAB)
    gather = _make_sc_gather()

    gs = [
        gather(lax.slice(idx_flat, (k * CROWS,), ((k + 1) * CROWS,)), token_emb)
        .reshape(CBATCH, SEQ, EMB)
        for k in range(NCHUNK)
    ]

    steps = CBATCH // _NB
    g_spec = pl.BlockSpec((_NB, SEQ, EMB), lambda i: (i, 0, 0))
    wt_spec = pl.BlockSpec((EMB, VOCAB), lambda i: (0, 0))
    b_spec = pl.BlockSpec((1, VOCAB), lambda i: (0, 0))

    parts = []
    for k in range(NCHUNK):
        out_spec = pl.BlockSpec((_NB * EMB, VOCAB), lambda i: (i, 0))
        part = pl.pallas_call(
            _tc_head_first,
            grid=(steps,),
            in_specs=[g_spec, wt_spec, b_spec],
            out_specs=out_spec,
            out_shape=jax.ShapeDtypeStruct((CROWS, VOCAB), jnp.float32),
        )(gs[k], wt, b2)
        parts.append(part.reshape(CBATCH, EMB, VOCAB))
    return jnp.concatenate(parts, axis=0)


# speedup vs baseline: 1.3551x; 1.3551x over previous
import functools

import jax
import jax.numpy as jnp
from jax import lax
from jax.experimental import pallas as pl
from jax.experimental.pallas import tpu as pltpu
from jax.experimental.pallas import tpu_sc as plsc

VOCAB = 1000
EMB = 128
BATCH = 1024
SEQ = 128
ROWS = BATCH * SEQ
NCHUNK = 4
CROWS = ROWS // NCHUNK
CBATCH = BATCH // NCHUNK

_NC = 2
_NS = 16
_NW = _NC * _NS
_BPW = CROWS // _NW
_CH = 128
_NCH = _BPW // _CH


@functools.cache
def _make_sc_gather():
    mesh = plsc.VectorSubcoreMesh(core_axis_name="c", subcore_axis_name="s")

    @functools.partial(
        pl.kernel,
        mesh=mesh,
        out_type=jax.ShapeDtypeStruct((CROWS, EMB), jnp.float32),
        scratch_types=[
            pltpu.VMEM((_BPW,), jnp.int32),
            pltpu.VMEM((2, _CH, EMB), jnp.float32),
            pltpu.SemaphoreType.DMA,
            pltpu.SemaphoreType.DMA,
            pltpu.SemaphoreType.DMA,
        ],
    )
    def _sc_gather(idx_hbm, table_hbm, out_hbm, idx_v, rows_v, gsem, ws0, ws1):
        wid = lax.axis_index("s") * _NC + lax.axis_index("c")
        base = wid * _BPW
        pltpu.sync_copy(idx_hbm.at[pl.ds(base, _BPW)], idx_v)

        def body2(g, carry):
            for slot in (0, 1):
                c = 2 * g + slot
                off = pl.multiple_of(c * _CH, _CH)
                wsem = ws0 if slot == 0 else ws1

                @pl.when(g > 0)
                def _wait_prev():
                    pltpu.make_async_copy(
                        rows_v.at[slot], out_hbm.at[pl.ds(base, _CH)], wsem
                    ).wait()

                pltpu.async_copy(
                    table_hbm.at[idx_v.at[pl.ds(off, _CH)]],
                    rows_v.at[slot], gsem,
                ).wait()
                pltpu.async_copy(
                    rows_v.at[slot], out_hbm.at[pl.ds(base + off, _CH)], wsem
                )
            return carry

        lax.fori_loop(0, _NCH // 2, body2, 0)
        pltpu.make_async_copy(
            rows_v.at[0], out_hbm.at[pl.ds(base, _CH)], ws0
        ).wait()
        pltpu.make_async_copy(
            rows_v.at[1], out_hbm.at[pl.ds(base, _CH)], ws1
        ).wait()

    return _sc_gather


_NB = 32


def _tc_head_first(g_ref, wt_ref, b_ref, out_ref):
    wt = wt_ref[...]
    bb = b_ref[...]
    for nb in range(_NB):
        g = g_ref[nb].astype(jnp.bfloat16)
        acc = lax.dot_general(
            g, wt, (((0,), (0,)), ((), ())),
            preferred_element_type=jnp.float32,
        )
        out_ref[nb] = acc + bb


def _tc_head_next(g_ref, wt_ref, b_ref, prev_ref, out_ref):
    _tc_head_first(g_ref, wt_ref, b_ref, out_ref)


def kernel(idxs, token_emb, W, b):
    idx_flat = idxs.reshape(-1).astype(jnp.int32)
    wt = W.astype(jnp.bfloat16).T
    b2 = b.reshape(1, VOCAB)
    gather = _make_sc_gather()

    gs = [
        gather(lax.slice(idx_flat, (k * CROWS,), ((k + 1) * CROWS,)), token_emb)
        .reshape(CBATCH, SEQ, EMB)
        for k in range(NCHUNK)
    ]

    steps = CBATCH // _NB
    g_spec = pl.BlockSpec((_NB, SEQ, EMB), lambda i: (i, 0, 0))
    wt_spec = pl.BlockSpec((EMB, VOCAB), lambda i: (0, 0))
    b_spec = pl.BlockSpec((1, VOCAB), lambda i: (0, 0))

    out = None
    for k in range(NCHUNK):
        out_spec = pl.BlockSpec(
            (_NB, EMB, VOCAB),
            functools.partial(lambda kk, i: (i + kk * steps, 0, 0), k),
        )
        if k == 0:
            out = pl.pallas_call(
                _tc_head_first,
                grid=(steps,),
                in_specs=[g_spec, wt_spec, b_spec],
                out_specs=out_spec,
                out_shape=jax.ShapeDtypeStruct((BATCH, EMB, VOCAB), jnp.float32),
            )(gs[0], wt, b2)
        else:
            out = pl.pallas_call(
                _tc_head_next,
                grid=(steps,),
                in_specs=[g_spec, wt_spec, b_spec,
                          pl.BlockSpec(memory_space=pl.ANY)],
                out_specs=out_spec,
                out_shape=jax.ShapeDtypeStruct((BATCH, EMB, VOCAB), jnp.float32),
                input_output_aliases={3: 0},
            )(gs[k], wt, b2, out)
    return out


# speedup vs baseline: 1.5489x; 1.1431x over previous
import functools

import jax
import jax.numpy as jnp
from jax import lax
from jax.experimental import pallas as pl
from jax.experimental.pallas import tpu as pltpu
from jax.experimental.pallas import tpu_sc as plsc

VOCAB = 1000
EMB = 128
BATCH = 1024
SEQ = 128
ROWS = BATCH * SEQ
NCHUNK = 4
CROWS = ROWS // NCHUNK
CBATCH = BATCH // NCHUNK

_NC = 2
_NS = 16
_NW = _NC * _NS
_BPW = CROWS // _NW
_CH = 128
_NCH = _BPW // _CH


@functools.cache
def _make_sc_gather():
    mesh = plsc.VectorSubcoreMesh(core_axis_name="c", subcore_axis_name="s")

    @functools.partial(
        pl.kernel,
        mesh=mesh,
        out_type=jax.ShapeDtypeStruct((CROWS, EMB), jnp.float32),
        scratch_types=[
            pltpu.VMEM((_BPW,), jnp.int32),
            pltpu.VMEM((2, _CH, EMB), jnp.float32),
            pltpu.SemaphoreType.DMA,
            pltpu.SemaphoreType.DMA,
            pltpu.SemaphoreType.DMA,
        ],
    )
    def _sc_gather(idx_hbm, table_hbm, out_hbm, idx_v, rows_v, gsem, ws0, ws1):
        wid = lax.axis_index("s") * _NC + lax.axis_index("c")
        base = wid * _BPW
        pltpu.sync_copy(idx_hbm.at[pl.ds(base, _BPW)], idx_v)

        def body2(g, carry):
            for slot in (0, 1):
                c = 2 * g + slot
                off = pl.multiple_of(c * _CH, _CH)
                wsem = ws0 if slot == 0 else ws1

                @pl.when(g > 0)
                def _wait_prev():
                    pltpu.make_async_copy(
                        rows_v.at[slot], out_hbm.at[pl.ds(base, _CH)], wsem
                    ).wait()

                pltpu.async_copy(
                    table_hbm.at[idx_v.at[pl.ds(off, _CH)]],
                    rows_v.at[slot], gsem,
                ).wait()
                pltpu.async_copy(
                    rows_v.at[slot], out_hbm.at[pl.ds(base + off, _CH)], wsem
                )
            return carry

        lax.fori_loop(0, _NCH // 2, body2, 0)
        pltpu.make_async_copy(
            rows_v.at[0], out_hbm.at[pl.ds(base, _CH)], ws0
        ).wait()
        pltpu.make_async_copy(
            rows_v.at[1], out_hbm.at[pl.ds(base, _CH)], ws1
        ).wait()

    return _sc_gather


_NB = 32


def _tc_head_first(g_ref, wt_ref, b_ref, out_ref):
    wt = wt_ref[...]
    bb = b_ref[...]
    for nb in range(_NB):
        g = g_ref[nb].astype(jnp.bfloat16)
        acc = lax.dot_general(
            g, wt, (((0,), (0,)), ((), ())),
            preferred_element_type=jnp.float32,
        )
        out_ref[pl.ds(nb * EMB, EMB), :] = acc + bb


def _tc_head_next(g_ref, wt_ref, b_ref, prev_ref, out_ref):
    _tc_head_first(g_ref, wt_ref, b_ref, out_ref)


def kernel(idxs, token_emb, W, b):
    idx_flat = idxs.reshape(-1).astype(jnp.int32)
    wt = W.astype(jnp.bfloat16).T
    b2 = b.reshape(1, VOCAB)
    gather = _make_sc_gather()

    gs = [
        gather(lax.slice(idx_flat, (k * CROWS,), ((k + 1) * CROWS,)), token_emb)
        .reshape(CBATCH, SEQ, EMB)
        for k in range(NCHUNK)
    ]

    steps = CBATCH // _NB
    g_spec = pl.BlockSpec((_NB, SEQ, EMB), lambda i: (i, 0, 0))
    wt_spec = pl.BlockSpec((EMB, VOCAB), lambda i: (0, 0))
    b_spec = pl.BlockSpec((1, VOCAB), lambda i: (0, 0))

    out2d = None
    for k in range(NCHUNK):
        out_spec = pl.BlockSpec(
            (_NB * EMB, VOCAB),
            functools.partial(lambda kk, i: (i + kk * steps, 0), k),
        )
        if k == 0:
            out2d = pl.pallas_call(
                _tc_head_first,
                grid=(steps,),
                in_specs=[g_spec, wt_spec, b_spec],
                out_specs=out_spec,
                out_shape=jax.ShapeDtypeStruct((ROWS, VOCAB), jnp.float32),
            )(gs[0], wt, b2)
        else:
            out2d = pl.pallas_call(
                _tc_head_next,
                grid=(steps,),
                in_specs=[g_spec, wt_spec, b_spec,
                          pl.BlockSpec(memory_space=pl.ANY)],
                out_specs=out_spec,
                out_shape=jax.ShapeDtypeStruct((ROWS, VOCAB), jnp.float32),
                input_output_aliases={3: 0},
            )(gs[k], wt, b2, out2d)
    return out2d.reshape(BATCH, EMB, VOCAB)
